# baseline (speedup 1.0000x reference)
import functools

import jax
import jax.numpy as jnp
from jax import lax
from jax.experimental import pallas as pl
from jax.experimental.pallas import tpu as pltpu
from jax.experimental.pallas import tpu_sc as plsc

DEPTH = 1000
BATCH = 4096
HIST = 20
NC = 2
NS = 16
L = 16
NW = NC * NS
BPW = BATCH // NW
CH = 4
NCHUNK = BPW // CH
ROWS_PER_CHUNK = CH * HIST
GROUPS = ROWS_PER_CHUNK // L
CHUNK_ELEMS = ROWS_PER_CHUNK * DEPTH

_mesh = plsc.VectorSubcoreMesh(core_axis_name="c", subcore_axis_name="s")


@functools.partial(
    pl.kernel,
    mesh=_mesh,
    out_type=jax.ShapeDtypeStruct((BATCH * HIST * DEPTH,), jnp.float32),
    scratch_types=[
        pltpu.VMEM((BPW * HIST,), jnp.int32),
        pltpu.VMEM((CHUNK_ELEMS,), jnp.float32),
    ],
    compiler_params=pltpu.CompilerParams(needs_layout_passes=False),
)
def _sc_onehot(idx_hbm, out_hbm, idx_v, buf):
    wid = lax.axis_index("s") * NC + lax.axis_index("c")
    base_b = wid * BPW
    pltpu.sync_copy(idx_hbm.at[pl.ds(base_b * HIST, BPW * HIST)], idx_v)

    zeros16 = jnp.zeros((L,), jnp.float32)
    ones16 = jnp.full((L,), 1.0, jnp.float32)
    iota16 = lax.iota(jnp.int32, L)

    def zbody(i, carry):
        buf[pl.ds(i * L, L)] = zeros16
        return carry

    lax.fori_loop(0, CHUNK_ELEMS // L, zbody, 0)

    def scatter_chunk(c, value_vec):
        for g in range(GROUPS):
            lr = g * L + iota16
            d16 = idx_v[pl.ds(c * ROWS_PER_CHUNK + g * L, L)]
            plsc.store_scatter(buf, [lr * DEPTH + d16], value_vec)

    def chunk_body(c, carry):
        scatter_chunk(c, ones16)
        out_off = (base_b + c * CH) * HIST * DEPTH
        pltpu.sync_copy(buf, out_hbm.at[pl.ds(out_off, CHUNK_ELEMS)])
        scatter_chunk(c, zeros16)
        return carry

    lax.fori_loop(0, NCHUNK, chunk_body, 0)


def kernel(input, emb_weight):
    del emb_weight
    flat = _sc_onehot(input.reshape(-1))
    return flat.reshape(BATCH, HIST, DEPTH)


# speedup vs baseline: 1.0089x; 1.0089x over previous
import functools

import jax
import jax.numpy as jnp
from jax import lax
from jax.experimental import pallas as pl
from jax.experimental.pallas import tpu as pltpu
from jax.experimental.pallas import tpu_sc as plsc

DEPTH = 1000
BATCH = 4096
HIST = 20
NC = 2
NS = 16
L = 16
NW = NC * NS
BPW = BATCH // NW
CH = 2
NCHUNK = BPW // CH
ROWS_PER_CHUNK = CH * HIST
FULL_GROUPS = ROWS_PER_CHUNK // L
TAIL = ROWS_PER_CHUNK - FULL_GROUPS * L
CHUNK_ELEMS = ROWS_PER_CHUNK * DEPTH
IDX_PAD = BPW * HIST + L

_mesh = plsc.VectorSubcoreMesh(core_axis_name="c", subcore_axis_name="s")


@functools.partial(
    pl.kernel,
    mesh=_mesh,
    out_type=jax.ShapeDtypeStruct((BATCH * HIST * DEPTH,), jnp.float32),
    scratch_types=[
        pltpu.VMEM((IDX_PAD,), jnp.int32),
        pltpu.VMEM((CHUNK_ELEMS,), jnp.float32),
        pltpu.VMEM((CHUNK_ELEMS,), jnp.float32),
        pltpu.SemaphoreType.DMA,
        pltpu.SemaphoreType.DMA,
    ],
    compiler_params=pltpu.CompilerParams(needs_layout_passes=False),
)
def _sc_onehot(idx_hbm, out_hbm, idx_v, buf0, buf1, sem0, sem1):
    bufs = (buf0, buf1)
    sems = (sem0, sem1)
    wid = lax.axis_index("s") * NC + lax.axis_index("c")
    base_b = wid * BPW
    pltpu.sync_copy(
        idx_hbm.at[pl.ds(base_b * HIST, BPW * HIST)],
        idx_v.at[pl.ds(0, BPW * HIST)],
    )

    zeros16 = jnp.zeros((L,), jnp.float32)
    ones16 = jnp.full((L,), 1.0, jnp.float32)
    iota16 = lax.iota(jnp.int32, L)
    tail_mask = iota16 < TAIL

    def zbody(i, carry):
        buf0[pl.ds(i * L, L)] = zeros16
        buf1[pl.ds(i * L, L)] = zeros16
        return carry

    lax.fori_loop(0, CHUNK_ELEMS // L, zbody, 0)

    def scatter_chunk(c, buf, value_vec):
        for g in range(FULL_GROUPS + 1):
            lr = g * L + iota16
            d16 = idx_v[pl.ds(c * ROWS_PER_CHUNK + g * L, L)]
            mask = None if g < FULL_GROUPS else tail_mask
            plsc.store_scatter(buf, [lr * DEPTH + d16], value_vec, mask=mask)

    def start_dma(c, buf, sem):
        out_off = (base_b + c * CH) * HIST * DEPTH
        pltpu.async_copy(buf, out_hbm.at[pl.ds(out_off, CHUNK_ELEMS)], sem)

    def drain(c, buf, sem):
        out_off = (base_b + c * CH) * HIST * DEPTH
        pltpu.make_async_copy(
            buf, out_hbm.at[pl.ds(out_off, CHUNK_ELEMS)], sem
        ).wait()

    for s in range(2):
        scatter_chunk(s, bufs[s], ones16)
        start_dma(s, bufs[s], sems[s])

    def pair_body(i, carry):
        c0 = 2 + 2 * i
        for s in range(2):
            c = c0 + s
            drain(c - 2, bufs[s], sems[s])
            scatter_chunk(c - 2, bufs[s], zeros16)
            scatter_chunk(c, bufs[s], ones16)
            start_dma(c, bufs[s], sems[s])
        return carry

    lax.fori_loop(0, (NCHUNK - 2) // 2, pair_body, 0)

    for s in range(2):
        drain(NCHUNK - 2 + s, bufs[s], sems[s])


def kernel(input, emb_weight):
    del emb_weight
    flat = _sc_onehot(input.reshape(-1))
    return flat.reshape(BATCH, HIST, DEPTH)


# speedup vs baseline: 1.4950x; 1.4819x over previous
import functools

import jax
import jax.numpy as jnp
from jax import lax
from jax.experimental import pallas as pl
from jax.experimental.pallas import tpu as pltpu
from jax.experimental.pallas import tpu_sc as plsc

DEPTH = 1000
BATCH = 4096
HIST = 20
NC = 2
NS = 16
L = 16
NW = NC * NS
BPW = BATCH // NW
CH = 2
NCHUNK = BPW // CH
ROWS_PER_CHUNK = CH * HIST
FULL_GROUPS = ROWS_PER_CHUNK // L
TAIL = ROWS_PER_CHUNK - FULL_GROUPS * L
IDX_PAD = BPW * HIST + L

_mesh = plsc.VectorSubcoreMesh(core_axis_name="c", subcore_axis_name="s")


@functools.partial(
    pl.kernel,
    mesh=_mesh,
    out_type=jax.ShapeDtypeStruct((BATCH, HIST, DEPTH), jnp.float32),
    scratch_types=[
        pltpu.VMEM((IDX_PAD,), jnp.int32),
        pltpu.VMEM((CH, HIST, DEPTH), jnp.float32),
        pltpu.VMEM((CH, HIST, DEPTH), jnp.float32),
        pltpu.SemaphoreType.DMA,
        pltpu.SemaphoreType.DMA,
    ],
    compiler_params=pltpu.CompilerParams(needs_layout_passes=False),
)
def _sc_onehot(idx_hbm, out_hbm, idx_v, buf0, buf1, sem0, sem1):
    bufs = (buf0, buf1)
    sems = (sem0, sem1)
    wid = lax.axis_index("s") * NC + lax.axis_index("c")
    base_b = wid * BPW
    pltpu.sync_copy(
        idx_hbm.at[pl.ds(base_b * HIST, BPW * HIST)],
        idx_v.at[pl.ds(0, BPW * HIST)],
    )

    zeros16 = jnp.zeros((L,), jnp.float32)
    ones16 = jnp.full((L,), 1.0, jnp.float32)
    iota16 = lax.iota(jnp.int32, L)
    tail_mask = iota16 < TAIL

    def zbody(i, carry):
        b16 = jnp.broadcast_to(i // HIST, (L,))
        h16 = jnp.broadcast_to(i % HIST, (L,))
        for buf in bufs:
            for g in range(DEPTH // L):
                plsc.store_scatter(buf, [b16, h16, g * L + iota16], zeros16)
            plsc.store_scatter(
                buf, [b16, h16, (DEPTH - L) + iota16], zeros16
            )
        return carry

    lax.fori_loop(0, ROWS_PER_CHUNK, zbody, 0)

    def scatter_chunk(c, buf, value_vec):
        for g in range(FULL_GROUPS + 1):
            lr = g * L + iota16
            b16 = lr // HIST
            h16 = lr % HIST
            d16 = idx_v[pl.ds(c * ROWS_PER_CHUNK + g * L, L)]
            mask = None if g < FULL_GROUPS else tail_mask
            plsc.store_scatter(buf, [b16, h16, d16], value_vec, mask=mask)

    def start_dma(c, buf, sem):
        pltpu.async_copy(buf, out_hbm.at[pl.ds(base_b + c * CH, CH)], sem)

    def drain(c, buf, sem):
        pltpu.make_async_copy(
            buf, out_hbm.at[pl.ds(base_b + c * CH, CH)], sem
        ).wait()

    for s in range(2):
        scatter_chunk(s, bufs[s], ones16)
        start_dma(s, bufs[s], sems[s])

    def pair_body(i, carry):
        c0 = 2 + 2 * i
        for s in range(2):
            c = c0 + s
            drain(c - 2, bufs[s], sems[s])
            scatter_chunk(c - 2, bufs[s], zeros16)
            scatter_chunk(c, bufs[s], ones16)
            start_dma(c, bufs[s], sems[s])
        return carry

    lax.fori_loop(0, (NCHUNK - 2) // 2, pair_body, 0)

    for s in range(2):
        drain(NCHUNK - 2 + s, bufs[s], sems[s])


def kernel(input, emb_weight):
    del emb_weight
    return _sc_onehot(input.reshape(-1))


# speedup vs baseline: 5.7387x; 3.8385x over previous
import functools

import jax
import jax.numpy as jnp
from jax import lax
from jax.experimental import pallas as pl
from jax.experimental.pallas import tpu as pltpu
from jax.experimental.pallas import tpu_sc as plsc

DEPTH = 1000
BATCH = 4096
HIST = 20
NC = 2
NS = 16
L = 16
NW = NC * NS
BPW = BATCH // NW
BGROUPS = BPW // L
DCH = 200
QN = DEPTH // DCH
NCHUNK = HIST * QN

_mesh = plsc.VectorSubcoreMesh(core_axis_name="c", subcore_axis_name="s")


@functools.partial(
    pl.kernel,
    mesh=_mesh,
    out_type=jax.ShapeDtypeStruct((HIST, DEPTH, BATCH), jnp.float32),
    scratch_types=[
        pltpu.VMEM((BPW * HIST,), jnp.int32),
        pltpu.VMEM((HIST * BPW,), jnp.int32),
        pltpu.VMEM((1, DCH, BPW), jnp.float32),
        pltpu.VMEM((1, DCH, BPW), jnp.float32),
        pltpu.SemaphoreType.DMA,
        pltpu.SemaphoreType.DMA,
    ],
    compiler_params=pltpu.CompilerParams(needs_layout_passes=False),
)
def _sc_onehot(idx_hbm, out_hbm, idx_v, idx_t, buf0, buf1, sem0, sem1):
    bufs = (buf0, buf1)
    sems = (sem0, sem1)
    wid = lax.axis_index("s") * NC + lax.axis_index("c")
    base_b = wid * BPW
    pltpu.sync_copy(idx_hbm.at[pl.ds(base_b * HIST, BPW * HIST)], idx_v)

    zeros16 = jnp.zeros((L,), jnp.float32)
    ones16 = jnp.full((L,), 1.0, jnp.float32)
    iota16 = lax.iota(jnp.int32, L)

    for h in range(HIST):
        for g in range(BGROUPS):
            b16 = g * L + iota16
            vals = plsc.load_gather(idx_v, [b16 * HIST + h])
            idx_t[pl.ds(h * BPW + g * L, L)] = vals

    def zbody(r, carry):
        for buf in bufs:
            for g in range(BGROUPS):
                buf[0, r, pl.ds(g * L, L)] = zeros16
        return carry

    lax.fori_loop(0, DCH, zbody, 0)

    zero16i = jnp.zeros((L,), jnp.int32)

    def scatter_chunk(h, q, buf, value_vec):
        d0 = q * DCH
        for g in range(BGROUPS):
            b16 = g * L + iota16
            d16 = idx_t[pl.ds(h * BPW + g * L, L)]
            dloc = d16 - d0
            mask = (dloc >= 0) & (dloc < DCH)
            plsc.store_scatter(buf, [zero16i, dloc, b16], value_vec, mask=mask)

    def start_dma(h, q, buf, sem):
        dst = out_hbm.at[pl.ds(h, 1), pl.ds(q * DCH, DCH), pl.ds(base_b, BPW)]
        pltpu.async_copy(buf, dst, sem)

    def drain(h, q, buf, sem):
        dst = out_hbm.at[pl.ds(h, 1), pl.ds(q * DCH, DCH), pl.ds(base_b, BPW)]
        pltpu.make_async_copy(buf, dst, sem).wait()

    for s in range(2):
        scatter_chunk(s // QN, s % QN, bufs[s], ones16)
        start_dma(s // QN, s % QN, bufs[s], sems[s])

    def pair_body(i, carry):
        c0 = 2 + 2 * i
        for s in range(2):
            c = c0 + s
            hp, qp = (c - 2) // QN, (c - 2) % QN
            h, q = c // QN, c % QN
            drain(hp, qp, bufs[s], sems[s])
            scatter_chunk(hp, qp, bufs[s], zeros16)
            scatter_chunk(h, q, bufs[s], ones16)
            start_dma(h, q, bufs[s], sems[s])
        return carry

    lax.fori_loop(0, (NCHUNK - 2) // 2, pair_body, 0)

    for s in range(2):
        c = NCHUNK - 2 + s
        drain(c // QN, c % QN, bufs[s], sems[s])


def kernel(input, emb_weight):
    del emb_weight
    out_t = _sc_onehot(input.reshape(-1))
    return out_t.transpose(2, 0, 1)
